# baseline (speedup 1.0000x reference)
import jax
import jax.numpy as jnp
from jax import lax
from jax.experimental import pallas as pl
from jax.experimental.pallas import tpu as pltpu
from jax.experimental.pallas import tpu_sc as plsc

DIM = 128
BATCH = 4096
LANES = 16
NUM_CORES = 2
NUM_SUBCORES = 16
NUM_WORKERS = NUM_CORES * NUM_SUBCORES
ROWS_PER_WORKER = BATCH // NUM_WORKERS
CHUNKS = DIM // LANES
NBLK = 4
BLK = ROWS_PER_WORKER // NBLK
TC_BLK = 1024


def _tc_dots_body(v_ref, e_ref, we_ref, wv_ref, s_ref):
    s_ref[...] = (
        lax.dot_general(we_ref[...], e_ref[...], (((1,), (1,)), ((), ())),
                        preferred_element_type=jnp.float32,
                        precision=lax.Precision.HIGHEST)
        + lax.dot_general(wv_ref[...], v_ref[...], (((1,), (1,)), ((), ())),
                          preferred_element_type=jnp.float32,
                          precision=lax.Precision.HIGHEST))


def _sc_body(v_hbm, e_hbm, s_hbm, b_v_h, b_e_h,
             vo_hbm, eo_hbm, v_b, e_b, vo_b, eo_b, w_v, s_v,
             w_sem, in_sem0, in_sem1, out_sem0, out_sem1):
    wid = lax.axis_index("s") * NUM_CORES + lax.axis_index("c")
    base = wid * ROWS_PER_WORKER
    in_sems = [in_sem0, in_sem1]
    out_sems = [out_sem0, out_sem1]

    pre_copies = [
        pltpu.async_copy(b_v_h, w_v.at[0], w_sem),
        pltpu.async_copy(b_e_h, w_v.at[1], w_sem),
        pltpu.async_copy(s_hbm.at[:, pl.ds(base, ROWS_PER_WORKER)], s_v,
                         w_sem),
    ]

    def start_in(blk):
        slot = blk % 2
        rows = pl.ds(base + blk * BLK, BLK)
        return (pltpu.async_copy(v_hbm.at[rows], v_b.at[slot], in_sems[slot]),
                pltpu.async_copy(e_hbm.at[rows], e_b.at[slot], in_sems[slot]))

    in_flight = {b: start_in(b) for b in range(2)}

    for c in pre_copies:
        c.wait()
    bch = [[w_v[k, pl.ds(c * LANES, LANES)] for c in range(CHUNKS)]
           for k in range(2)]

    out_flight = {}
    for blk in range(NBLK):
        slot = blk % 2
        for h in in_flight.pop(blk):
            h.wait()
        if blk >= 2:
            for h in out_flight.pop(blk - 2):
                h.wait()

        rbase = blk * BLK

        @plsc.parallel_loop(0, BLK, step=LANES)
        def _(g):
            sv = [s_v[k, pl.ds(rbase + g, LANES)] for k in range(4)]
            for r16 in range(LANES):
                lane = jnp.full((LANES,), r16, jnp.int32)
                s_vv = sv[0].at[lane].get(mode="promise_in_bounds")
                s_ev = sv[1].at[lane].get(mode="promise_in_bounds")
                s_ve = sv[2].at[lane].get(mode="promise_in_bounds")
                s_ee = sv[3].at[lane].get(mode="promise_in_bounds")
                r = g + r16
                for c in range(CHUNKS):
                    sl = pl.ds(c * LANES, LANES)
                    vch = v_b[slot, r, sl]
                    ech = e_b[slot, r, sl]
                    vo_b[slot, r, sl] = vch * s_vv + ech * s_ev + bch[0][c]
                    eo_b[slot, r, sl] = vch * s_ve + ech * s_ee + bch[1][c]

        rows = pl.ds(base + blk * BLK, BLK)
        out_flight[blk] = (
            pltpu.async_copy(vo_b.at[slot], vo_hbm.at[rows], out_sems[slot]),
            pltpu.async_copy(eo_b.at[slot], eo_hbm.at[rows], out_sems[slot]))
        if blk + 2 < NBLK:
            in_flight[blk + 2] = start_in(blk + 2)

    for blk in (NBLK - 2, NBLK - 1):
        for h in out_flight.pop(blk):
            h.wait()


@jax.jit
def _run(v, e, w_e, w_v, b_v, b_e):
    s = pl.pallas_call(
        _tc_dots_body,
        out_shape=jax.ShapeDtypeStruct((8, BATCH), jnp.float32),
        grid=(BATCH // TC_BLK,),
        in_specs=[
            pl.BlockSpec((TC_BLK, DIM), lambda i: (i, 0)),
            pl.BlockSpec((TC_BLK, DIM), lambda i: (i, 0)),
            pl.BlockSpec((8, DIM), lambda i: (0, 0)),
            pl.BlockSpec((8, DIM), lambda i: (0, 0)),
        ],
        out_specs=pl.BlockSpec((8, TC_BLK), lambda i: (0, i)),
    )(v, e, w_e, w_v)

    mesh = plsc.VectorSubcoreMesh(
        core_axis_name="c", subcore_axis_name="s",
        num_cores=NUM_CORES, num_subcores=NUM_SUBCORES)
    run = pl.kernel(
        _sc_body,
        out_type=(
            jax.ShapeDtypeStruct((BATCH, DIM), jnp.float32),
            jax.ShapeDtypeStruct((BATCH, DIM), jnp.float32),
        ),
        mesh=mesh,
        compiler_params=pltpu.CompilerParams(needs_layout_passes=False),
        scratch_types=[
            pltpu.VMEM((2, BLK, DIM), jnp.float32),
            pltpu.VMEM((2, BLK, DIM), jnp.float32),
            pltpu.VMEM((2, BLK, DIM), jnp.float32),
            pltpu.VMEM((2, BLK, DIM), jnp.float32),
            pltpu.VMEM((2, DIM), jnp.float32),
            pltpu.VMEM((8, ROWS_PER_WORKER), jnp.float32),
            pltpu.SemaphoreType.DMA,
            pltpu.SemaphoreType.DMA,
            pltpu.SemaphoreType.DMA,
            pltpu.SemaphoreType.DMA,
            pltpu.SemaphoreType.DMA,
        ],
    )
    return run(v, e, s, b_v, b_e)


def kernel(v, e, w_vv, w_ev, w_ve, w_ee, b_v, b_e):
    zero = jnp.zeros((DIM,), jnp.float32)
    w_e = jnp.stack([w_vv.reshape(DIM), zero, w_ve.reshape(DIM), zero,
                     zero, zero, zero, zero])
    w_v = jnp.stack([zero, w_ev.reshape(DIM), zero, w_ee.reshape(DIM),
                     zero, zero, zero, zero])
    return _run(v, e, w_e, w_v, b_v, b_e)


# speedup vs baseline: 1.3428x; 1.3428x over previous
import jax
import jax.numpy as jnp
from jax import lax
from jax.experimental import pallas as pl
from jax.experimental.pallas import tpu as pltpu
from jax.experimental.pallas import tpu_sc as plsc

DIM = 128
BATCH = 4096
LANES = 16
NUM_CORES = 2
NUM_SUBCORES = 16
NUM_WORKERS = NUM_CORES * NUM_SUBCORES
ROWS_PER_WORKER = BATCH // NUM_WORKERS
CHUNKS = DIM // LANES
NBLK = 8
BLK = ROWS_PER_WORKER // NBLK


def _sc_body(v_hbm, e_hbm, w_vv_h, w_ev_h, w_ve_h, w_ee_h, b_v_h, b_e_h,
             vo_hbm, eo_hbm, v_b, e_b, vo_b, eo_b, w_v, s_v,
             w_sem, in_sem0, in_sem1, out_sem0, out_sem1):
    wid = lax.axis_index("s") * NUM_CORES + lax.axis_index("c")
    base = wid * ROWS_PER_WORKER
    in_sems = [in_sem0, in_sem1]
    out_sems = [out_sem0, out_sem1]

    w_copies = [
        pltpu.async_copy(h, w_v.at[k], w_sem)
        for k, h in enumerate([w_vv_h, w_ev_h, w_ve_h, w_ee_h, b_v_h, b_e_h])
    ]

    def start_in(blk):
        slot = blk % 2
        rows = pl.ds(base + blk * BLK, BLK)
        return (pltpu.async_copy(v_hbm.at[rows], v_b.at[slot], in_sems[slot]),
                pltpu.async_copy(e_hbm.at[rows], e_b.at[slot], in_sems[slot]))

    in_flight = {b: start_in(b) for b in range(2)}

    for c in w_copies:
        c.wait()
    wch = [[w_v[k, pl.ds(c * LANES, LANES)] for c in range(CHUNKS)]
           for k in range(6)]

    out_flight = {}
    for blk in range(NBLK):
        slot = blk % 2
        for h in in_flight.pop(blk):
            h.wait()
        if blk >= 2:
            for h in out_flight.pop(blk - 2):
                h.wait()

        @plsc.parallel_loop(0, BLK)
        def _(r):
            a_vv = jnp.zeros((LANES,), jnp.float32)
            a_ev = jnp.zeros((LANES,), jnp.float32)
            a_ve = jnp.zeros((LANES,), jnp.float32)
            a_ee = jnp.zeros((LANES,), jnp.float32)
            for c in range(CHUNKS):
                sl = pl.ds(c * LANES, LANES)
                vch = v_b[slot, r, sl]
                ech = e_b[slot, r, sl]
                a_vv = a_vv + ech * wch[0][c]
                a_ev = a_ev + vch * wch[1][c]
                a_ve = a_ve + ech * wch[2][c]
                a_ee = a_ee + vch * wch[3][c]
            s_v[0, r] = jnp.sum(a_vv)
            s_v[1, r] = jnp.sum(a_ev)
            s_v[2, r] = jnp.sum(a_ve)
            s_v[3, r] = jnp.sum(a_ee)

        @plsc.parallel_loop(0, BLK)
        def _(r):
            s_vv = s_v[0, r]
            s_ev = s_v[1, r]
            s_ve = s_v[2, r]
            s_ee = s_v[3, r]
            for c in range(CHUNKS):
                sl = pl.ds(c * LANES, LANES)
                vch = v_b[slot, r, sl]
                ech = e_b[slot, r, sl]
                vo_b[slot, r, sl] = vch * s_vv + ech * s_ev + wch[4][c]
                eo_b[slot, r, sl] = vch * s_ve + ech * s_ee + wch[5][c]

        rows = pl.ds(base + blk * BLK, BLK)
        out_flight[blk] = (
            pltpu.async_copy(vo_b.at[slot], vo_hbm.at[rows], out_sems[slot]),
            pltpu.async_copy(eo_b.at[slot], eo_hbm.at[rows], out_sems[slot]))
        if blk + 2 < NBLK:
            in_flight[blk + 2] = start_in(blk + 2)

    for blk in (NBLK - 2, NBLK - 1):
        for h in out_flight.pop(blk):
            h.wait()


@jax.jit
def _sc_call(v, e, w_vv, w_ev, w_ve, w_ee, b_v, b_e):
    mesh = plsc.VectorSubcoreMesh(
        core_axis_name="c", subcore_axis_name="s",
        num_cores=NUM_CORES, num_subcores=NUM_SUBCORES)
    run = pl.kernel(
        _sc_body,
        out_type=(
            jax.ShapeDtypeStruct((BATCH, DIM), jnp.float32),
            jax.ShapeDtypeStruct((BATCH, DIM), jnp.float32),
        ),
        mesh=mesh,
        compiler_params=pltpu.CompilerParams(needs_layout_passes=False),
        scratch_types=[
            pltpu.VMEM((2, BLK, DIM), jnp.float32),
            pltpu.VMEM((2, BLK, DIM), jnp.float32),
            pltpu.VMEM((2, BLK, DIM), jnp.float32),
            pltpu.VMEM((2, BLK, DIM), jnp.float32),
            pltpu.VMEM((6, DIM), jnp.float32),
            pltpu.SMEM((4, BLK), jnp.float32),
            pltpu.SemaphoreType.DMA,
            pltpu.SemaphoreType.DMA,
            pltpu.SemaphoreType.DMA,
            pltpu.SemaphoreType.DMA,
            pltpu.SemaphoreType.DMA,
        ],
    )
    return run(v, e, w_vv, w_ev, w_ve, w_ee, b_v, b_e)


def kernel(v, e, w_vv, w_ev, w_ve, w_ee, b_v, b_e):
    return _sc_call(v, e, w_vv.reshape(DIM), w_ev.reshape(DIM),
                    w_ve.reshape(DIM), w_ee.reshape(DIM), b_v, b_e)


# speedup vs baseline: 1.5405x; 1.1472x over previous
import jax
import jax.numpy as jnp
from jax import lax
from jax.experimental import pallas as pl
from jax.experimental.pallas import tpu as pltpu
from jax.experimental.pallas import tpu_sc as plsc

DIM = 128
BATCH = 4096
LANES = 16
NUM_CORES = 2
NUM_SUBCORES = 16
NUM_WORKERS = NUM_CORES * NUM_SUBCORES
ROWS_PER_WORKER = BATCH // NUM_WORKERS
CHUNKS = DIM // LANES
NBLK = 2
BLK = ROWS_PER_WORKER // NBLK


def _sc_body(v_hbm, e_hbm, w_vv_h, w_ev_h, w_ve_h, w_ee_h, b_v_h, b_e_h,
             vo_hbm, eo_hbm, v_b, e_b, vo_b, eo_b, w_v, s_v,
             w_sem, in_sem0, in_sem1, out_sem0, out_sem1):
    wid = lax.axis_index("s") * NUM_CORES + lax.axis_index("c")
    base = wid * ROWS_PER_WORKER
    in_sems = [in_sem0, in_sem1]
    out_sems = [out_sem0, out_sem1]

    w_copies = [
        pltpu.async_copy(h, w_v.at[k], w_sem)
        for k, h in enumerate([w_vv_h, w_ev_h, w_ve_h, w_ee_h, b_v_h, b_e_h])
    ]

    def start_in(blk):
        slot = blk % 2
        rows = pl.ds(base + blk * BLK, BLK)
        return (pltpu.async_copy(v_hbm.at[rows], v_b.at[slot], in_sems[slot]),
                pltpu.async_copy(e_hbm.at[rows], e_b.at[slot], in_sems[slot]))

    in_flight = {b: start_in(b) for b in range(2)}

    for c in w_copies:
        c.wait()
    wch = [[w_v[k, pl.ds(c * LANES, LANES)] for c in range(CHUNKS)]
           for k in range(6)]

    out_flight = {}
    for blk in range(NBLK):
        slot = blk % 2
        for h in in_flight.pop(blk):
            h.wait()
        if blk >= 2:
            for h in out_flight.pop(blk - 2):
                h.wait()

        @plsc.parallel_loop(0, BLK)
        def _(r):
            a_vv = jnp.zeros((LANES,), jnp.float32)
            a_ev = jnp.zeros((LANES,), jnp.float32)
            a_ve = jnp.zeros((LANES,), jnp.float32)
            a_ee = jnp.zeros((LANES,), jnp.float32)
            for c in range(CHUNKS):
                sl = pl.ds(c * LANES, LANES)
                vch = v_b[slot, r, sl]
                ech = e_b[slot, r, sl]
                a_vv = a_vv + ech * wch[0][c]
                a_ev = a_ev + vch * wch[1][c]
                a_ve = a_ve + ech * wch[2][c]
                a_ee = a_ee + vch * wch[3][c]
            s_v[0, r] = jnp.sum(a_vv)
            s_v[1, r] = jnp.sum(a_ev)
            s_v[2, r] = jnp.sum(a_ve)
            s_v[3, r] = jnp.sum(a_ee)

        @plsc.parallel_loop(0, BLK)
        def _(r):
            s_vv = s_v[0, r]
            s_ev = s_v[1, r]
            s_ve = s_v[2, r]
            s_ee = s_v[3, r]
            for c in range(CHUNKS):
                sl = pl.ds(c * LANES, LANES)
                vch = v_b[slot, r, sl]
                ech = e_b[slot, r, sl]
                vo_b[slot, r, sl] = vch * s_vv + ech * s_ev + wch[4][c]
                eo_b[slot, r, sl] = vch * s_ve + ech * s_ee + wch[5][c]

        rows = pl.ds(base + blk * BLK, BLK)
        out_flight[blk] = (
            pltpu.async_copy(vo_b.at[slot], vo_hbm.at[rows], out_sems[slot]),
            pltpu.async_copy(eo_b.at[slot], eo_hbm.at[rows], out_sems[slot]))
        if blk + 2 < NBLK:
            in_flight[blk + 2] = start_in(blk + 2)

    for blk in (NBLK - 2, NBLK - 1):
        for h in out_flight.pop(blk):
            h.wait()


@jax.jit
def _sc_call(v, e, w_vv, w_ev, w_ve, w_ee, b_v, b_e):
    mesh = plsc.VectorSubcoreMesh(
        core_axis_name="c", subcore_axis_name="s",
        num_cores=NUM_CORES, num_subcores=NUM_SUBCORES)
    run = pl.kernel(
        _sc_body,
        out_type=(
            jax.ShapeDtypeStruct((BATCH, DIM), jnp.float32),
            jax.ShapeDtypeStruct((BATCH, DIM), jnp.float32),
        ),
        mesh=mesh,
        compiler_params=pltpu.CompilerParams(needs_layout_passes=False),
        scratch_types=[
            pltpu.VMEM((2, BLK, DIM), jnp.float32),
            pltpu.VMEM((2, BLK, DIM), jnp.float32),
            pltpu.VMEM((2, BLK, DIM), jnp.float32),
            pltpu.VMEM((2, BLK, DIM), jnp.float32),
            pltpu.VMEM((6, DIM), jnp.float32),
            pltpu.SMEM((4, BLK), jnp.float32),
            pltpu.SemaphoreType.DMA,
            pltpu.SemaphoreType.DMA,
            pltpu.SemaphoreType.DMA,
            pltpu.SemaphoreType.DMA,
            pltpu.SemaphoreType.DMA,
        ],
    )
    return run(v, e, w_vv, w_ev, w_ve, w_ee, b_v, b_e)


def kernel(v, e, w_vv, w_ev, w_ve, w_ee, b_v, b_e):
    return _sc_call(v, e, w_vv.reshape(DIM), w_ev.reshape(DIM),
                    w_ve.reshape(DIM), w_ee.reshape(DIM), b_v, b_e)
